# Initial kernel scaffold; baseline (speedup 1.0000x reference)
import jax
import jax.numpy as jnp
from jax.experimental import pallas as pl


def kernel(x, table):
    raise NotImplementedError("write your pallas kernel here")


# speedup vs baseline: 6.3490x; 6.3490x over previous
import functools

import jax
import jax.numpy as jnp
from jax import lax
from jax.experimental import pallas as pl
from jax.experimental.pallas import tpu as pltpu
from jax.experimental.pallas import tpu_sc as plsc

VOCAB = 100000
EMBED_DIM = 128
BATCH = 4096
HIST = 200

N = BATCH * HIST
NC, NS = 2, 16
NW = NC * NS
PER_W = N // NW
CHUNK = 128
NCHUNK = PER_W // CHUNK

_mesh = plsc.VectorSubcoreMesh(core_axis_name="c", subcore_axis_name="s")


@functools.partial(
    pl.kernel,
    out_type=jax.ShapeDtypeStruct((N, EMBED_DIM), jnp.float32),
    mesh=_mesh,
    scratch_types=[
        pltpu.VMEM((NCHUNK, CHUNK), jnp.int32),
        pltpu.VMEM((CHUNK, EMBED_DIM), jnp.float32),
        pltpu.SemaphoreType.DMA,
    ],
)
def _gather_kernel(x_hbm, table_hbm, out_hbm, idx_v, rows_v, sem):
    wid = lax.axis_index("s") * NC + lax.axis_index("c")
    pltpu.sync_copy(x_hbm.at[pl.ds(wid * NCHUNK, NCHUNK)], idx_v)
    base = wid * PER_W

    def body(j, carry):
        pltpu.async_copy(table_hbm.at[idx_v.at[j]], rows_v, sem).wait()
        pltpu.sync_copy(rows_v, out_hbm.at[pl.ds(base + j * CHUNK, CHUNK)])
        return carry

    lax.fori_loop(0, NCHUNK, body, 0)


def kernel(x, table):
    xf = x.reshape(-1).astype(jnp.int32).reshape(N // CHUNK, CHUNK)
    out = _gather_kernel(xf, table)
    return out.reshape(BATCH, HIST, EMBED_DIM)


# speedup vs baseline: 9.0764x; 1.4296x over previous
import functools

import jax
import jax.numpy as jnp
from jax import lax
from jax.experimental import pallas as pl
from jax.experimental.pallas import tpu as pltpu
from jax.experimental.pallas import tpu_sc as plsc

VOCAB = 100000
EMBED_DIM = 128
BATCH = 4096
HIST = 200

N = BATCH * HIST
NC, NS = 2, 16
NW = NC * NS
PER_W = N // NW
CHUNK = 128
NCHUNK = PER_W // CHUNK
K = 2
NBUF = 2 * K
NPAIR = NCHUNK // (2 * K)

_mesh = plsc.VectorSubcoreMesh(core_axis_name="c", subcore_axis_name="s")


@functools.partial(
    pl.kernel,
    out_type=jax.ShapeDtypeStruct((N, EMBED_DIM), jnp.float32),
    mesh=_mesh,
    scratch_types=[
        pltpu.VMEM((NCHUNK, CHUNK), jnp.int32),
        pltpu.VMEM((NBUF, CHUNK, EMBED_DIM), jnp.float32),
        pltpu.SemaphoreType.DMA((NBUF,)),
        pltpu.SemaphoreType.DMA((NBUF,)),
    ],
)
def _gather_kernel(x_hbm, table_hbm, out_hbm, idx_v, rows_v, gsem, ssem):
    wid = lax.axis_index("s") * NC + lax.axis_index("c")
    pltpu.sync_copy(x_hbm.at[pl.ds(wid * NCHUNK, NCHUNK)], idx_v)
    base = wid * PER_W

    def g_start(j, b):
        pltpu.async_copy(table_hbm.at[idx_v.at[j]], rows_v.at[b], gsem.at[b])

    def g_wait(b):
        pltpu.make_async_copy(
            table_hbm.at[idx_v.at[0]], rows_v.at[b], gsem.at[b]
        ).wait()

    def s_start(j, b):
        pltpu.async_copy(
            rows_v.at[b], out_hbm.at[pl.ds(base + j * CHUNK, CHUNK)], ssem.at[b]
        )

    def s_wait(b):
        pltpu.make_async_copy(
            rows_v.at[b], out_hbm.at[pl.ds(0, CHUNK)], ssem.at[b]
        ).wait()

    for b in range(NBUF):
        g_start(b, b)

    def body(t, carry):
        c0 = t * (2 * K)
        for b in range(K):
            g_wait(b)
        for b in range(K):
            s_start(c0 + b, b)
        for b in range(K):
            g_wait(K + b)
        for b in range(K):
            s_start(c0 + K + b, K + b)
        for b in range(K):
            s_wait(b)
            g_start(c0 + 2 * K + b, b)
        for b in range(K):
            s_wait(K + b)
            g_start(c0 + 3 * K + b, K + b)
        return carry

    lax.fori_loop(0, NPAIR - 1, body, 0)

    c0 = (NPAIR - 1) * (2 * K)
    for b in range(K):
        g_wait(b)
    for b in range(K):
        s_start(c0 + b, b)
    for b in range(K):
        g_wait(K + b)
    for b in range(K):
        s_start(c0 + K + b, K + b)
    for b in range(NBUF):
        s_wait(b)


def kernel(x, table):
    xf = x.reshape(-1).astype(jnp.int32).reshape(N // CHUNK, CHUNK)
    out = _gather_kernel(xf, table)
    return out.reshape(BATCH, HIST, EMBED_DIM)


# speedup vs baseline: 9.1802x; 1.0114x over previous
import functools

import jax
import jax.numpy as jnp
from jax import lax
from jax.experimental import pallas as pl
from jax.experimental.pallas import tpu as pltpu
from jax.experimental.pallas import tpu_sc as plsc

VOCAB = 100000
EMBED_DIM = 128
BATCH = 4096
HIST = 200

N = BATCH * HIST
NC, NS = 2, 16
NW = NC * NS
PER_W = N // NW
CHUNK = 80
NCHUNK = PER_W // CHUNK
K = 4
NBUF = 2 * K
NPAIR = NCHUNK // (2 * K)

_mesh = plsc.VectorSubcoreMesh(core_axis_name="c", subcore_axis_name="s")


@functools.partial(
    pl.kernel,
    out_type=jax.ShapeDtypeStruct((N, EMBED_DIM), jnp.float32),
    mesh=_mesh,
    scratch_types=[
        pltpu.VMEM((NCHUNK, CHUNK), jnp.int32),
        pltpu.VMEM((NBUF, CHUNK, EMBED_DIM), jnp.float32),
        pltpu.SemaphoreType.DMA((NBUF,)),
        pltpu.SemaphoreType.DMA((NBUF,)),
    ],
)
def _gather_kernel(x_hbm, table_hbm, out_hbm, idx_v, rows_v, gsem, ssem):
    wid = lax.axis_index("s") * NC + lax.axis_index("c")
    pltpu.sync_copy(x_hbm.at[pl.ds(wid * NCHUNK, NCHUNK)], idx_v)
    base = wid * PER_W

    def g_start(j, b):
        pltpu.async_copy(table_hbm.at[idx_v.at[j]], rows_v.at[b], gsem.at[b])

    def g_wait(b):
        pltpu.make_async_copy(
            table_hbm.at[idx_v.at[0]], rows_v.at[b], gsem.at[b]
        ).wait()

    def s_start(j, b):
        pltpu.async_copy(
            rows_v.at[b], out_hbm.at[pl.ds(base + j * CHUNK, CHUNK)], ssem.at[b]
        )

    def s_wait(b):
        pltpu.make_async_copy(
            rows_v.at[b], out_hbm.at[pl.ds(0, CHUNK)], ssem.at[b]
        ).wait()

    for b in range(NBUF):
        g_start(b, b)

    def body(t, carry):
        c0 = t * (2 * K)
        for b in range(K):
            g_wait(b)
        for b in range(K):
            s_start(c0 + b, b)
        for b in range(K):
            g_wait(K + b)
        for b in range(K):
            s_start(c0 + K + b, K + b)
        for b in range(K):
            s_wait(b)
            g_start(c0 + 2 * K + b, b)
        for b in range(K):
            s_wait(K + b)
            g_start(c0 + 3 * K + b, K + b)
        return carry

    lax.fori_loop(0, NPAIR - 1, body, 0)

    c0 = (NPAIR - 1) * (2 * K)
    for b in range(K):
        g_wait(b)
    for b in range(K):
        s_start(c0 + b, b)
    for b in range(K):
        g_wait(K + b)
    for b in range(K):
        s_start(c0 + K + b, K + b)
    for b in range(NBUF):
        s_wait(b)


def kernel(x, table):
    xf = x.reshape(-1).astype(jnp.int32).reshape(N // CHUNK, CHUNK)
    out = _gather_kernel(xf, table)
    return out.reshape(BATCH, HIST, EMBED_DIM)
